# Initial kernel scaffold; baseline (speedup 1.0000x reference)
import jax
import jax.numpy as jnp
from jax.experimental import pallas as pl


def kernel(x, edge_index, batch, Wl1, bl1, Wr1, br1, att1, bias1, Wl2, bl2, Wr2, br2, att2, bias2, Wp, bp):
    raise NotImplementedError("write your pallas kernel here")


# speedup vs baseline: 38.1351x; 38.1351x over previous
import functools

import jax
import jax.numpy as jnp
from jax import lax
from jax.experimental import pallas as pl
from jax.experimental.pallas import tpu as pltpu
from jax.experimental.pallas import tpu_sc as plsc

N = 10000
E = 320000
DIN = 128
HID = 64
HEADS = 4
NG = 64

NC = 2
NS = 16
NW = NC * NS
L = 16

CH = 128
NCHUNK = E // CH
TBL = 128
NP_ = 10240
RPT = NP_ // NS

_BN = 400
_GRID = N // _BN


def _proj_body(x_ref, wl_ref, bl_ref, wr_ref, br_ref, xlr_ref):
    xb = x_ref[...]
    xlr_ref[:, :HID] = jnp.dot(xb, wl_ref[...], preferred_element_type=jnp.float32) + bl_ref[...]
    xlr_ref[:, HID:] = jnp.dot(xb, wr_ref[...], preferred_element_type=jnp.float32) + br_ref[...]


def _tc_proj(x, Wl, bl, Wr, br):
    din = x.shape[1]
    return pl.pallas_call(
        _proj_body,
        grid=(_GRID,),
        in_specs=[
            pl.BlockSpec((_BN, din), lambda i: (i, 0)),
            pl.BlockSpec((din, HID), lambda i: (0, 0)),
            pl.BlockSpec((1, HID), lambda i: (0, 0)),
            pl.BlockSpec((din, HID), lambda i: (0, 0)),
            pl.BlockSpec((1, HID), lambda i: (0, 0)),
        ],
        out_specs=pl.BlockSpec((_BN, 2 * HID), lambda i: (i, 0)),
        out_shape=jax.ShapeDtypeStruct((N, 2 * HID), jnp.float32),
    )(x, Wl, bl.reshape(1, HID), Wr, br.reshape(1, HID))


def _sc_edge_kernel(nheads, xlr, att, src, dst):
    mesh = plsc.VectorSubcoreMesh(core_axis_name="c", subcore_axis_name="s")

    @functools.partial(
        pl.kernel,
        mesh=mesh,
        out_type=jax.ShapeDtypeStruct((NC, NP_, TBL), jnp.float32),
        scratch_types=[
            pltpu.VMEM_SHARED((NP_, TBL), jnp.float32),
            pltpu.VMEM((RPT // 10, TBL), jnp.float32),
            pltpu.VMEM((CH,), jnp.int32),
            pltpu.VMEM((CH,), jnp.int32),
            pltpu.VMEM((CH, TBL), jnp.float32),
            pltpu.VMEM((CH, TBL), jnp.float32),
            pltpu.VMEM((HID,), jnp.float32),
            pltpu.SemaphoreType.DMA,
            pltpu.SemaphoreType.DMA,
        ],
    )
    def k(xlr_hbm, att_hbm, src_hbm, dst_hbm, part_hbm,
          tbl_sh, zbuf, sidx, didx, gsrc, gdst, attv, sem0, sem1):
        cid = lax.axis_index("c")
        sid = lax.axis_index("s")
        wid = sid * NC + cid

        zero = jnp.zeros((L,), jnp.float32)

        def zrow(r, carry):
            for j in range(TBL // L):
                zbuf[r, pl.ds(j * L, L)] = zero
            return carry

        lax.fori_loop(0, RPT // 10, zrow, 0)
        for q in range(10):
            pltpu.sync_copy(zbuf, tbl_sh.at[pl.ds(sid * RPT + q * (RPT // 10), RPT // 10)])

        pltpu.sync_copy(att_hbm, attv)
        plsc.subcore_barrier()

        iot = lax.iota(jnp.int32, L)
        avs = [attv[pl.ds(h * L, L)] for h in range(HEADS)]

        gdn = lax.GatherDimensionNumbers(
            offset_dims=(), collapsed_slice_dims=(0,), start_index_map=(0,))
        perm = [(iot ^ (1 << b))[:, None] for b in range(4)]

        def _allsum(v):
            for b in range(4):
                v = v + lax.gather(v, perm[b], gdn, slice_sizes=(1,),
                                   mode=lax.GatherScatterMode.PROMISE_IN_BOUNDS)
            return v

        nch = NCHUNK // NW + jnp.where(wid < NCHUNK % NW, 1, 0)

        def chunk_body(kk, carry):
            base = (wid + kk * NW) * CH
            pltpu.sync_copy(src_hbm.at[pl.ds(base, CH)], sidx)
            pltpu.sync_copy(dst_hbm.at[pl.ds(base, CH)], didx)
            cp0 = pltpu.async_copy(xlr_hbm.at[sidx], gsrc, sem0)
            cp1 = pltpu.async_copy(xlr_hbm.at[didx], gdst, sem1)
            cp0.wait()
            cp1.wait()

            if nheads > 1:
                def edge_body(e, c2):
                    dvec = jnp.zeros((L,), jnp.float32)
                    for h in range(HEADS):
                        s = gsrc[e, pl.ds(h * L, L)]
                        d = gdst[e, pl.ds(HID + h * L, L)]
                        eh = s + d
                        eh = jnp.maximum(eh, 0.2 * eh)
                        pb = jnp.exp(_allsum(eh * avs[h]))
                        gsrc[e, pl.ds(h * L, L)] = pb * s
                        dvec = jnp.where(iot == h, pb, dvec)
                    gsrc[e, pl.ds(HID, L)] = dvec
                    return c2
            else:
                def edge_body(e, c2):
                    svecs = []
                    am = jnp.zeros((L,), jnp.float32)
                    for h in range(HEADS):
                        s = gsrc[e, pl.ds(h * L, L)]
                        d = gdst[e, pl.ds(HID + h * L, L)]
                        eh = s + d
                        eh = jnp.maximum(eh, 0.2 * eh)
                        am = am + eh * avs[h]
                        svecs.append(s)
                    pb = jnp.exp(_allsum(am))
                    for h in range(HEADS):
                        gsrc[e, pl.ds(h * L, L)] = pb * svecs[h]
                    gsrc[e, pl.ds(HID, L)] = jnp.where(iot == 0, pb, zero)
                    return c2

            lax.fori_loop(0, CH, edge_body, 0)
            pltpu.sync_copy(gsrc, tbl_sh.at[didx], add=True)
            return carry

        lax.fori_loop(0, nch, chunk_body, 0)
        plsc.subcore_barrier()

        pltpu.sync_copy(tbl_sh.at[pl.ds(sid * RPT, RPT)],
                        part_hbm.at[cid, pl.ds(sid * RPT, RPT)])

    return k(xlr, att, src, dst)


def _mid_body(part_ref, b1_ref, wl_ref, bl_ref, wr_ref, br_ref, hlr_ref):
    p0 = part_ref[0]
    p1 = part_ref[1]
    acc = p0[:, :HID] + p1[:, :HID]
    den = p0[:, HID:HID + HEADS] + p1[:, HID:HID + HEADS]
    hsel = (lax.broadcasted_iota(jnp.int32, (HEADS, HID), 1) // (HID // HEADS)
            == lax.broadcasted_iota(jnp.int32, (HEADS, HID), 0)).astype(jnp.float32)
    dene = jnp.dot(den, hsel, preferred_element_type=jnp.float32)
    h = jnp.maximum(acc / (dene + 1e-16) + b1_ref[...], 0.0)
    hlr_ref[:, :HID] = jnp.dot(h, wl_ref[...], preferred_element_type=jnp.float32) + bl_ref[...]
    hlr_ref[:, HID:] = jnp.dot(h, wr_ref[...], preferred_element_type=jnp.float32) + br_ref[...]


def _tc_mid(part, bias1, Wl2, bl2, Wr2, br2):
    return pl.pallas_call(
        _mid_body,
        grid=(_GRID,),
        in_specs=[
            pl.BlockSpec((NC, _BN, TBL), lambda i: (0, i, 0)),
            pl.BlockSpec((1, HID), lambda i: (0, 0)),
            pl.BlockSpec((HID, HID), lambda i: (0, 0)),
            pl.BlockSpec((1, HID), lambda i: (0, 0)),
            pl.BlockSpec((HID, HID), lambda i: (0, 0)),
            pl.BlockSpec((1, HID), lambda i: (0, 0)),
        ],
        out_specs=pl.BlockSpec((_BN, 2 * HID), lambda i: (i, 0)),
        out_shape=jax.ShapeDtypeStruct((N, 2 * HID), jnp.float32),
    )(part, bias1.reshape(1, HID), Wl2, bl2.reshape(1, HID), Wr2, br2.reshape(1, HID))


def _post_body(part_ref, b2_ref, batch_ref, wp_ref, bp_ref, out_ref,
               pooled_s, counts_s):
    i = pl.program_id(0)
    p0 = part_ref[0]
    p1 = part_ref[1]
    den = p0[:, HID:HID + 1] + p1[:, HID:HID + 1]
    h2 = jnp.maximum((p0[:, :HID] + p1[:, :HID]) / (den + 1e-16) + b2_ref[...], 0.0)
    bcol = batch_ref[...]
    P = (bcol == lax.broadcasted_iota(jnp.int32, (_BN, NG), 1)).astype(jnp.float32)
    pp = lax.dot_general(P, h2, (((0,), (0,)), ((), ())),
                         preferred_element_type=jnp.float32)
    cc = lax.dot_general(P, jnp.ones((_BN, 1), jnp.float32), (((0,), (0,)), ((), ())),
                         preferred_element_type=jnp.float32)

    @pl.when(i == 0)
    def _():
        pooled_s[...] = pp
        counts_s[...] = cc

    @pl.when(i > 0)
    def _():
        pooled_s[...] += pp
        counts_s[...] += cc

    @pl.when(i == _GRID - 1)
    def _():
        pooled = pooled_s[...] / jnp.maximum(counts_s[...], 1.0)
        out_ref[...] = jnp.dot(pooled, wp_ref[...], preferred_element_type=jnp.float32) + bp_ref[...]


def _tc_post(part, bias2, batch, Wp, bp):
    return pl.pallas_call(
        _post_body,
        grid=(_GRID,),
        in_specs=[
            pl.BlockSpec((NC, _BN, TBL), lambda i: (0, i, 0)),
            pl.BlockSpec((1, HID), lambda i: (0, 0)),
            pl.BlockSpec((_BN, 1), lambda i: (i, 0)),
            pl.BlockSpec((HID, 1), lambda i: (0, 0)),
            pl.BlockSpec((1, 1), lambda i: (0, 0)),
        ],
        out_specs=pl.BlockSpec((NG, 1), lambda i: (0, 0)),
        out_shape=jax.ShapeDtypeStruct((NG, 1), jnp.float32),
        scratch_shapes=[
            pltpu.VMEM((NG, NG), jnp.float32),
            pltpu.VMEM((NG, 1), jnp.float32),
        ],
    )(part, bias2.reshape(1, HID), batch.reshape(N, 1), Wp, bp.reshape(1, 1))


def kernel(x, edge_index, batch, Wl1, bl1, Wr1, br1, att1, bias1,
           Wl2, bl2, Wr2, br2, att2, bias2, Wp, bp):
    src = edge_index[0]
    dst = edge_index[1]

    xlr1 = _tc_proj(x, Wl1, bl1, Wr1, br1)
    part1 = _sc_edge_kernel(HEADS, xlr1, att1.reshape(HID), src, dst)
    hlr2 = _tc_mid(part1, bias1, Wl2, bl2, Wr2, br2)
    part2 = _sc_edge_kernel(1, hlr2, att2.reshape(HID), src, dst)
    out = _tc_post(part2, bias2, batch, Wp, bp)
    return out.reshape(NG)


# speedup vs baseline: 59.7961x; 1.5680x over previous
import functools

import jax
import jax.numpy as jnp
from jax import lax
from jax.experimental import pallas as pl
from jax.experimental.pallas import tpu as pltpu
from jax.experimental.pallas import tpu_sc as plsc

N = 10000
E = 320000
DIN = 128
HID = 64
HEADS = 4
NG = 64

NC = 2
NS = 16
NW = NC * NS
L = 16

CH = 80
NCHUNK = E // CH
NCHT = NCHUNK // NW
TBL = 128
NP_ = 10240
RPT = NP_ // NS
ZR = 16

_BN = 400
_GRID = N // _BN


def _proj_body(x_ref, wl_ref, bl_ref, wr_ref, br_ref, xlr_ref):
    xb = x_ref[...]
    xlr_ref[:, :HID] = jnp.dot(xb, wl_ref[...], preferred_element_type=jnp.float32) + bl_ref[...]
    xlr_ref[:, HID:] = jnp.dot(xb, wr_ref[...], preferred_element_type=jnp.float32) + br_ref[...]


def _tc_proj(x, Wl, bl, Wr, br):
    din = x.shape[1]
    return pl.pallas_call(
        _proj_body,
        grid=(_GRID,),
        in_specs=[
            pl.BlockSpec((_BN, din), lambda i: (i, 0)),
            pl.BlockSpec((din, HID), lambda i: (0, 0)),
            pl.BlockSpec((1, HID), lambda i: (0, 0)),
            pl.BlockSpec((din, HID), lambda i: (0, 0)),
            pl.BlockSpec((1, HID), lambda i: (0, 0)),
        ],
        out_specs=pl.BlockSpec((_BN, 2 * HID), lambda i: (i, 0)),
        out_shape=jax.ShapeDtypeStruct((N, 2 * HID), jnp.float32),
    )(x, Wl, bl.reshape(1, HID), Wr, br.reshape(1, HID))


def _sc_edge_kernel(nheads, xlr, att, src2, dst2):
    mesh = plsc.VectorSubcoreMesh(core_axis_name="c", subcore_axis_name="s")

    @functools.partial(
        pl.kernel,
        mesh=mesh,
        out_type=jax.ShapeDtypeStruct((NC, NP_, TBL), jnp.float32),
        scratch_types=[
            pltpu.VMEM_SHARED((NP_, TBL), jnp.float32),
            pltpu.VMEM((ZR, TBL), jnp.float32),
            pltpu.VMEM((CH,), jnp.int32),
            pltpu.VMEM((CH,), jnp.int32),
            pltpu.VMEM((CH,), jnp.int32),
            pltpu.VMEM((CH,), jnp.int32),
            pltpu.VMEM((CH,), jnp.int32),
            pltpu.VMEM((CH,), jnp.int32),
            pltpu.VMEM((CH,), jnp.int32),
            pltpu.VMEM((CH,), jnp.int32),
            pltpu.VMEM((CH, TBL), jnp.float32),
            pltpu.VMEM((CH, TBL), jnp.float32),
            pltpu.VMEM((CH, TBL), jnp.float32),
            pltpu.VMEM((CH, TBL), jnp.float32),
            pltpu.VMEM((HID,), jnp.float32),
        ] + [pltpu.SemaphoreType.DMA] * 15,
    )
    def k(xlr_hbm, att_hbm, src_hbm, dst_hbm, part_hbm,
          tbl_sh, zbuf, si0, si1, si2, si3, di0, di1, di2, di3,
          gs0, gs1, gd0, gd1, attv,
          zsem, ssi0, ssi1, ssi2, ssi3, sdi0, sdi1, sdi2, sdi3,
          sgs0, sgs1, sgd0, sgd1, ssc0, ssc1):
        cid = lax.axis_index("c")
        sid = lax.axis_index("s")
        wid = sid * NC + cid
        c0 = wid * NCHT

        SI = (si0, si1, si2, si3)
        DI = (di0, di1, di2, di3)
        GS = (gs0, gs1)
        GD = (gd0, gd1)
        SSI = (ssi0, ssi1, ssi2, ssi3)
        SDI = (sdi0, sdi1, sdi2, sdi3)
        SGS = (sgs0, sgs1)
        SGD = (sgd0, sgd1)
        SSC = (ssc0, ssc1)

        zero = jnp.zeros((L,), jnp.float32)

        def zrow(r, carry):
            for j in range(TBL // L):
                zbuf[r, pl.ds(j * L, L)] = zero
            return carry

        lax.fori_loop(0, ZR, zrow, 0)
        for r in range(RPT // ZR // 8):
            zcps = [pltpu.async_copy(
                        zbuf, tbl_sh.at[pl.ds(sid * RPT + (r * 8 + q) * ZR, ZR)], zsem)
                    for q in range(8)]
            for cp in zcps:
                cp.wait()
        pltpu.sync_copy(att_hbm, attv)
        plsc.subcore_barrier()

        iot = lax.iota(jnp.int32, L)
        avs = [attv[pl.ds(h * L, L)] for h in range(HEADS)]

        gdn = lax.GatherDimensionNumbers(
            offset_dims=(), collapsed_slice_dims=(0,), start_index_map=(0,))
        perm = [(iot ^ (1 << b))[:, None] for b in range(4)]

        def _allsum(v):
            for b in range(4):
                v = v + lax.gather(v, perm[b], gdn, slice_sizes=(1,),
                                   mode=lax.GatherScatterMode.PROMISE_IN_BOUNDS)
            return v

        if nheads > 1:
            def edge_body_for(gsrc, gdst):
                def edge_body(e, c2):
                    dvec = jnp.zeros((L,), jnp.float32)
                    for h in range(HEADS):
                        s = gsrc[e, pl.ds(h * L, L)]
                        d = gdst[e, pl.ds(HID + h * L, L)]
                        eh = s + d
                        eh = jnp.maximum(eh, 0.2 * eh)
                        pb = jnp.exp(_allsum(eh * avs[h]))
                        gsrc[e, pl.ds(h * L, L)] = pb * s
                        dvec = jnp.where(iot == h, pb, dvec)
                    gsrc[e, pl.ds(HID, L)] = dvec
                    return c2
                return edge_body
        else:
            def edge_body_for(gsrc, gdst):
                def edge_body(e, c2):
                    svecs = []
                    am = jnp.zeros((L,), jnp.float32)
                    for h in range(HEADS):
                        s = gsrc[e, pl.ds(h * L, L)]
                        d = gdst[e, pl.ds(HID + h * L, L)]
                        eh = s + d
                        eh = jnp.maximum(eh, 0.2 * eh)
                        am = am + eh * avs[h]
                        svecs.append(s)
                    pb = jnp.exp(_allsum(am))
                    for h in range(HEADS):
                        gsrc[e, pl.ds(h * L, L)] = pb * svecs[h]
                    gsrc[e, pl.ds(HID, L)] = jnp.where(iot == 0, pb, zero)
                    return c2
                return edge_body

        pltpu.async_copy(src_hbm.at[c0], SI[0], SSI[0])
        pltpu.async_copy(dst_hbm.at[c0], DI[0], SDI[0])
        pltpu.async_copy(src_hbm.at[c0 + 1], SI[1], SSI[1])
        pltpu.async_copy(dst_hbm.at[c0 + 1], DI[1], SDI[1])
        pltpu.make_async_copy(src_hbm.at[c0], SI[0], SSI[0]).wait()
        pltpu.make_async_copy(dst_hbm.at[c0], DI[0], SDI[0]).wait()
        pltpu.async_copy(xlr_hbm.at[SI[0]], GS[0], SGS[0])
        pltpu.async_copy(xlr_hbm.at[DI[0]], GD[0], SGD[0])

        def quad_body(p, carry):
            for b in range(4):
                S, T = b % 2, 1 - b % 2
                i4 = b
                j4 = (b + 1) % 4
                n4 = (b + 2) % 4
                w4 = (b + 3) % 4
                c = 4 * p + b

                @pl.when(c < NCHT)
                def _():
                    @pl.when(c + 1 < NCHT)
                    def _():
                        @pl.when(c >= 1)
                        def _():
                            pltpu.make_async_copy(
                                GS[T], tbl_sh.at[DI[w4]], SSC[T]).wait()
                        pltpu.make_async_copy(src_hbm.at[c0 + c + 1], SI[j4], SSI[j4]).wait()
                        pltpu.make_async_copy(dst_hbm.at[c0 + c + 1], DI[j4], SDI[j4]).wait()
                        pltpu.async_copy(xlr_hbm.at[SI[j4]], GS[T], SGS[T])
                        pltpu.async_copy(xlr_hbm.at[DI[j4]], GD[T], SGD[T])

                    pltpu.make_async_copy(xlr_hbm.at[SI[i4]], GS[S], SGS[S]).wait()
                    pltpu.make_async_copy(xlr_hbm.at[DI[i4]], GD[S], SGD[S]).wait()

                    @pl.when(c + 2 < NCHT)
                    def _():
                        pltpu.async_copy(src_hbm.at[c0 + c + 2], SI[n4], SSI[n4])
                        pltpu.async_copy(dst_hbm.at[c0 + c + 2], DI[n4], SDI[n4])

                    lax.fori_loop(0, CH, edge_body_for(GS[S], GD[S]), 0)
                    pltpu.async_copy(GS[S], tbl_sh.at[DI[i4]], SSC[S], add=True)
            return carry

        lax.fori_loop(0, (NCHT + 3) // 4, quad_body, 0)

        pltpu.make_async_copy(GS[(NCHT - 2) % 2], tbl_sh.at[DI[(NCHT - 2) % 4]],
                              SSC[(NCHT - 2) % 2]).wait()
        pltpu.make_async_copy(GS[(NCHT - 1) % 2], tbl_sh.at[DI[(NCHT - 1) % 4]],
                              SSC[(NCHT - 1) % 2]).wait()
        plsc.subcore_barrier()

        pltpu.sync_copy(tbl_sh.at[pl.ds(sid * RPT, RPT)],
                        part_hbm.at[cid, pl.ds(sid * RPT, RPT)])

    return k(xlr, att, src2, dst2)


def _mid_body(part_ref, b1_ref, wl_ref, bl_ref, wr_ref, br_ref, hlr_ref):
    p0 = part_ref[0]
    p1 = part_ref[1]
    acc = p0[:, :HID] + p1[:, :HID]
    den = p0[:, HID:HID + HEADS] + p1[:, HID:HID + HEADS]
    hsel = (lax.broadcasted_iota(jnp.int32, (HEADS, HID), 1) // (HID // HEADS)
            == lax.broadcasted_iota(jnp.int32, (HEADS, HID), 0)).astype(jnp.float32)
    dene = jnp.dot(den, hsel, preferred_element_type=jnp.float32)
    h = jnp.maximum(acc / (dene + 1e-16) + b1_ref[...], 0.0)
    hlr_ref[:, :HID] = jnp.dot(h, wl_ref[...], preferred_element_type=jnp.float32) + bl_ref[...]
    hlr_ref[:, HID:] = jnp.dot(h, wr_ref[...], preferred_element_type=jnp.float32) + br_ref[...]


def _tc_mid(part, bias1, Wl2, bl2, Wr2, br2):
    return pl.pallas_call(
        _mid_body,
        grid=(_GRID,),
        in_specs=[
            pl.BlockSpec((NC, _BN, TBL), lambda i: (0, i, 0)),
            pl.BlockSpec((1, HID), lambda i: (0, 0)),
            pl.BlockSpec((HID, HID), lambda i: (0, 0)),
            pl.BlockSpec((1, HID), lambda i: (0, 0)),
            pl.BlockSpec((HID, HID), lambda i: (0, 0)),
            pl.BlockSpec((1, HID), lambda i: (0, 0)),
        ],
        out_specs=pl.BlockSpec((_BN, 2 * HID), lambda i: (i, 0)),
        out_shape=jax.ShapeDtypeStruct((N, 2 * HID), jnp.float32),
    )(part, bias1.reshape(1, HID), Wl2, bl2.reshape(1, HID), Wr2, br2.reshape(1, HID))


def _post_body(part_ref, b2_ref, batch_ref, wp_ref, bp_ref, out_ref,
               pooled_s, counts_s):
    i = pl.program_id(0)
    p0 = part_ref[0]
    p1 = part_ref[1]
    den = p0[:, HID:HID + 1] + p1[:, HID:HID + 1]
    h2 = jnp.maximum((p0[:, :HID] + p1[:, :HID]) / (den + 1e-16) + b2_ref[...], 0.0)
    bcol = batch_ref[...]
    P = (bcol == lax.broadcasted_iota(jnp.int32, (_BN, NG), 1)).astype(jnp.float32)
    pp = lax.dot_general(P, h2, (((0,), (0,)), ((), ())),
                         preferred_element_type=jnp.float32)
    cc = lax.dot_general(P, jnp.ones((_BN, 1), jnp.float32), (((0,), (0,)), ((), ())),
                         preferred_element_type=jnp.float32)

    @pl.when(i == 0)
    def _():
        pooled_s[...] = pp
        counts_s[...] = cc

    @pl.when(i > 0)
    def _():
        pooled_s[...] += pp
        counts_s[...] += cc

    @pl.when(i == _GRID - 1)
    def _():
        pooled = pooled_s[...] / jnp.maximum(counts_s[...], 1.0)
        out_ref[...] = jnp.dot(pooled, wp_ref[...], preferred_element_type=jnp.float32) + bp_ref[...]


def _tc_post(part, bias2, batch, Wp, bp):
    return pl.pallas_call(
        _post_body,
        grid=(_GRID,),
        in_specs=[
            pl.BlockSpec((NC, _BN, TBL), lambda i: (0, i, 0)),
            pl.BlockSpec((1, HID), lambda i: (0, 0)),
            pl.BlockSpec((_BN, 1), lambda i: (i, 0)),
            pl.BlockSpec((HID, 1), lambda i: (0, 0)),
            pl.BlockSpec((1, 1), lambda i: (0, 0)),
        ],
        out_specs=pl.BlockSpec((NG, 1), lambda i: (0, 0)),
        out_shape=jax.ShapeDtypeStruct((NG, 1), jnp.float32),
        scratch_shapes=[
            pltpu.VMEM((NG, NG), jnp.float32),
            pltpu.VMEM((NG, 1), jnp.float32),
        ],
    )(part, bias2.reshape(1, HID), batch.reshape(N, 1), Wp, bp.reshape(1, 1))


def kernel(x, edge_index, batch, Wl1, bl1, Wr1, br1, att1, bias1,
           Wl2, bl2, Wr2, br2, att2, bias2, Wp, bp):
    src = edge_index[0]
    dst = edge_index[1]

    src2 = src.reshape(NCHUNK, CH)
    dst2 = dst.reshape(NCHUNK, CH)
    xlr1 = _tc_proj(x, Wl1, bl1, Wr1, br1)
    part1 = _sc_edge_kernel(HEADS, xlr1, att1.reshape(HID), src2, dst2)
    hlr2 = _tc_mid(part1, bias1, Wl2, bl2, Wr2, br2)
    part2 = _sc_edge_kernel(1, hlr2, att2.reshape(HID), src2, dst2)
    out = _tc_post(part2, bias2, batch, Wp, bp)
    return out.reshape(NG)


# speedup vs baseline: 92.7146x; 1.5505x over previous
import functools

import jax
import jax.numpy as jnp
from jax import lax
from jax.experimental import pallas as pl
from jax.experimental.pallas import tpu as pltpu
from jax.experimental.pallas import tpu_sc as plsc

N = 10000
E = 320000
DIN = 128
HID = 64
HEADS = 4
NG = 64

NC = 2
NS = 16
NW = NC * NS
L = 16

CH = 80
NCHUNK = E // CH
NCHT = NCHUNK // NW
TBL = 128
NP_ = 10240
RPT = NP_ // NS
ZR = 16

_BN = 400
_GRID = N // _BN


def _proj_body(x_ref, wl_ref, bl_ref, wr_ref, br_ref, xlr_ref):
    xb = x_ref[...]
    xlr_ref[:, :HID] = jnp.dot(xb, wl_ref[...], preferred_element_type=jnp.float32) + bl_ref[...]
    xlr_ref[:, HID:] = jnp.dot(xb, wr_ref[...], preferred_element_type=jnp.float32) + br_ref[...]


def _tc_proj(x, Wl, bl, Wr, br):
    din = x.shape[1]
    return pl.pallas_call(
        _proj_body,
        grid=(_GRID,),
        in_specs=[
            pl.BlockSpec((_BN, din), lambda i: (i, 0)),
            pl.BlockSpec((din, HID), lambda i: (0, 0)),
            pl.BlockSpec((1, HID), lambda i: (0, 0)),
            pl.BlockSpec((din, HID), lambda i: (0, 0)),
            pl.BlockSpec((1, HID), lambda i: (0, 0)),
        ],
        out_specs=pl.BlockSpec((_BN, 2 * HID), lambda i: (i, 0)),
        out_shape=jax.ShapeDtypeStruct((N, 2 * HID), jnp.float32),
    )(x, Wl, bl.reshape(1, HID), Wr, br.reshape(1, HID))


def _sc_edge_kernel(nheads, xlr, att, src2, dst2):
    mesh = plsc.VectorSubcoreMesh(core_axis_name="c", subcore_axis_name="s")

    @functools.partial(
        pl.kernel,
        mesh=mesh,
        out_type=jax.ShapeDtypeStruct((NC, NP_, TBL), jnp.float32),
        scratch_types=[
            pltpu.VMEM_SHARED((NP_, TBL), jnp.float32),
            pltpu.VMEM((ZR, TBL), jnp.float32),
            pltpu.VMEM((CH,), jnp.int32),
            pltpu.VMEM((CH,), jnp.int32),
            pltpu.VMEM((CH,), jnp.int32),
            pltpu.VMEM((CH,), jnp.int32),
            pltpu.VMEM((CH,), jnp.int32),
            pltpu.VMEM((CH,), jnp.int32),
            pltpu.VMEM((CH,), jnp.int32),
            pltpu.VMEM((CH,), jnp.int32),
            pltpu.VMEM((CH, TBL), jnp.float32),
            pltpu.VMEM((CH, TBL), jnp.float32),
            pltpu.VMEM((CH, TBL), jnp.float32),
            pltpu.VMEM((CH, TBL), jnp.float32),
            pltpu.VMEM((HID,), jnp.float32),
        ] + [pltpu.SemaphoreType.DMA] * 15,
    )
    def k(xlr_hbm, att_hbm, src_hbm, dst_hbm, part_hbm,
          tbl_sh, zbuf, si0, si1, si2, si3, di0, di1, di2, di3,
          gs0, gs1, gd0, gd1, attv,
          zsem, ssi0, ssi1, ssi2, ssi3, sdi0, sdi1, sdi2, sdi3,
          sgs0, sgs1, sgd0, sgd1, ssc0, ssc1):
        cid = lax.axis_index("c")
        sid = lax.axis_index("s")
        wid = sid * NC + cid
        c0 = wid * NCHT

        SI = (si0, si1, si2, si3)
        DI = (di0, di1, di2, di3)
        GS = (gs0, gs1)
        GD = (gd0, gd1)
        SSI = (ssi0, ssi1, ssi2, ssi3)
        SDI = (sdi0, sdi1, sdi2, sdi3)
        SGS = (sgs0, sgs1)
        SGD = (sgd0, sgd1)
        SSC = (ssc0, ssc1)

        zero = jnp.zeros((L,), jnp.float32)

        def zrow(r, carry):
            for j in range(TBL // L):
                zbuf[r, pl.ds(j * L, L)] = zero
            return carry

        lax.fori_loop(0, ZR, zrow, 0)
        for r in range(RPT // ZR // 8):
            zcps = [pltpu.async_copy(
                        zbuf, tbl_sh.at[pl.ds(sid * RPT + (r * 8 + q) * ZR, ZR)], zsem)
                    for q in range(8)]
            for cp in zcps:
                cp.wait()
        pltpu.sync_copy(att_hbm, attv)
        plsc.subcore_barrier()

        iot = lax.iota(jnp.int32, L)
        avs = [attv[pl.ds(h * L, L)] for h in range(HEADS)]

        gdn = lax.GatherDimensionNumbers(
            offset_dims=(), collapsed_slice_dims=(0,), start_index_map=(0,))
        perm = [(iot ^ (1 << b))[:, None] for b in range(4)]

        def _allsum(v):
            for b in range(4):
                v = v + lax.gather(v, perm[b], gdn, slice_sizes=(1,),
                                   mode=lax.GatherScatterMode.PROMISE_IN_BOUNDS)
            return v

        if nheads > 1:
            def edge_body_for(gsrc, gdst):
                def edge_body(e, c2):
                    dvec = jnp.zeros((L,), jnp.float32)
                    for h in range(HEADS):
                        s = gsrc[e, pl.ds(h * L, L)]
                        d = gdst[e, pl.ds(HID + h * L, L)]
                        eh = s + d
                        eh = jnp.maximum(eh, 0.2 * eh)
                        pb = jnp.exp(_allsum(eh * avs[h]))
                        gsrc[e, pl.ds(h * L, L)] = pb * s
                        dvec = jnp.where(iot == h, pb, dvec)
                    gsrc[e, pl.ds(HID, L)] = dvec
                    return c2
                return edge_body
        else:
            def edge_body_for(gsrc, gdst):
                def edge_body(e, c2):
                    svecs = []
                    am = jnp.zeros((L,), jnp.float32)
                    for h in range(HEADS):
                        s = gsrc[e, pl.ds(h * L, L)]
                        d = gdst[e, pl.ds(HID + h * L, L)]
                        eh = s + d
                        eh = jnp.maximum(eh, 0.2 * eh)
                        am = am + eh * avs[h]
                        svecs.append(s)
                    pb = jnp.exp(_allsum(am))
                    for h in range(HEADS):
                        gsrc[e, pl.ds(h * L, L)] = pb * svecs[h]
                    gsrc[e, pl.ds(HID, L)] = jnp.where(iot == 0, pb, zero)
                    return c2
                return edge_body

        pltpu.async_copy(src_hbm.at[c0], SI[0], SSI[0])
        pltpu.async_copy(dst_hbm.at[c0], DI[0], SDI[0])
        pltpu.async_copy(src_hbm.at[c0 + 1], SI[1], SSI[1])
        pltpu.async_copy(dst_hbm.at[c0 + 1], DI[1], SDI[1])
        pltpu.make_async_copy(src_hbm.at[c0], SI[0], SSI[0]).wait()
        pltpu.make_async_copy(dst_hbm.at[c0], DI[0], SDI[0]).wait()
        pltpu.async_copy(xlr_hbm.at[SI[0]], GS[0], SGS[0])
        pltpu.async_copy(xlr_hbm.at[DI[0]], GD[0], SGD[0])

        def quad_body(p, carry):
            for b in range(4):
                S, T = b % 2, 1 - b % 2
                i4 = b
                j4 = (b + 1) % 4
                n4 = (b + 2) % 4
                w4 = (b + 3) % 4
                c = 4 * p + b

                @pl.when(c < NCHT)
                def _():
                    @pl.when(c + 1 < NCHT)
                    def _():
                        @pl.when(c >= 1)
                        def _():
                            pltpu.make_async_copy(
                                GS[T], tbl_sh.at[DI[w4]], SSC[T]).wait()
                        pltpu.make_async_copy(src_hbm.at[c0 + c + 1], SI[j4], SSI[j4]).wait()
                        pltpu.make_async_copy(dst_hbm.at[c0 + c + 1], DI[j4], SDI[j4]).wait()
                        pltpu.async_copy(xlr_hbm.at[SI[j4]], GS[T], SGS[T])
                        pltpu.async_copy(xlr_hbm.at[DI[j4]], GD[T], SGD[T])

                    pltpu.make_async_copy(xlr_hbm.at[SI[i4]], GS[S], SGS[S]).wait()
                    pltpu.make_async_copy(xlr_hbm.at[DI[i4]], GD[S], SGD[S]).wait()

                    @pl.when(c + 2 < NCHT)
                    def _():
                        pltpu.async_copy(src_hbm.at[c0 + c + 2], SI[n4], SSI[n4])
                        pltpu.async_copy(dst_hbm.at[c0 + c + 2], DI[n4], SDI[n4])

                    plsc.parallel_loop(0, CH, 1, unroll=4, carry=jnp.int32(0))(
                        edge_body_for(GS[S], GD[S]))
                    pltpu.async_copy(GS[S], tbl_sh.at[DI[i4]], SSC[S], add=True)
            return carry

        lax.fori_loop(0, (NCHT + 3) // 4, quad_body, 0)

        pltpu.make_async_copy(GS[(NCHT - 2) % 2], tbl_sh.at[DI[(NCHT - 2) % 4]],
                              SSC[(NCHT - 2) % 2]).wait()
        pltpu.make_async_copy(GS[(NCHT - 1) % 2], tbl_sh.at[DI[(NCHT - 1) % 4]],
                              SSC[(NCHT - 1) % 2]).wait()
        plsc.subcore_barrier()

        pltpu.sync_copy(tbl_sh.at[pl.ds(sid * RPT, RPT)],
                        part_hbm.at[cid, pl.ds(sid * RPT, RPT)])

    return k(xlr, att, src2, dst2)


def _mid_body(part_ref, b1_ref, wl_ref, bl_ref, wr_ref, br_ref, hlr_ref):
    p0 = part_ref[0]
    p1 = part_ref[1]
    acc = p0[:, :HID] + p1[:, :HID]
    den = p0[:, HID:HID + HEADS] + p1[:, HID:HID + HEADS]
    hsel = (lax.broadcasted_iota(jnp.int32, (HEADS, HID), 1) // (HID // HEADS)
            == lax.broadcasted_iota(jnp.int32, (HEADS, HID), 0)).astype(jnp.float32)
    dene = jnp.dot(den, hsel, preferred_element_type=jnp.float32)
    h = jnp.maximum(acc / (dene + 1e-16) + b1_ref[...], 0.0)
    hlr_ref[:, :HID] = jnp.dot(h, wl_ref[...], preferred_element_type=jnp.float32) + bl_ref[...]
    hlr_ref[:, HID:] = jnp.dot(h, wr_ref[...], preferred_element_type=jnp.float32) + br_ref[...]


def _tc_mid(part, bias1, Wl2, bl2, Wr2, br2):
    return pl.pallas_call(
        _mid_body,
        grid=(_GRID,),
        in_specs=[
            pl.BlockSpec((NC, _BN, TBL), lambda i: (0, i, 0)),
            pl.BlockSpec((1, HID), lambda i: (0, 0)),
            pl.BlockSpec((HID, HID), lambda i: (0, 0)),
            pl.BlockSpec((1, HID), lambda i: (0, 0)),
            pl.BlockSpec((HID, HID), lambda i: (0, 0)),
            pl.BlockSpec((1, HID), lambda i: (0, 0)),
        ],
        out_specs=pl.BlockSpec((_BN, 2 * HID), lambda i: (i, 0)),
        out_shape=jax.ShapeDtypeStruct((N, 2 * HID), jnp.float32),
    )(part, bias1.reshape(1, HID), Wl2, bl2.reshape(1, HID), Wr2, br2.reshape(1, HID))


def _post_body(part_ref, b2_ref, batch_ref, wp_ref, bp_ref, out_ref,
               pooled_s, counts_s):
    i = pl.program_id(0)
    p0 = part_ref[0]
    p1 = part_ref[1]
    den = p0[:, HID:HID + 1] + p1[:, HID:HID + 1]
    h2 = jnp.maximum((p0[:, :HID] + p1[:, :HID]) / (den + 1e-16) + b2_ref[...], 0.0)
    bcol = batch_ref[...]
    P = (bcol == lax.broadcasted_iota(jnp.int32, (_BN, NG), 1)).astype(jnp.float32)
    pp = lax.dot_general(P, h2, (((0,), (0,)), ((), ())),
                         preferred_element_type=jnp.float32)
    cc = lax.dot_general(P, jnp.ones((_BN, 1), jnp.float32), (((0,), (0,)), ((), ())),
                         preferred_element_type=jnp.float32)

    @pl.when(i == 0)
    def _():
        pooled_s[...] = pp
        counts_s[...] = cc

    @pl.when(i > 0)
    def _():
        pooled_s[...] += pp
        counts_s[...] += cc

    @pl.when(i == _GRID - 1)
    def _():
        pooled = pooled_s[...] / jnp.maximum(counts_s[...], 1.0)
        out_ref[...] = jnp.dot(pooled, wp_ref[...], preferred_element_type=jnp.float32) + bp_ref[...]


def _tc_post(part, bias2, batch, Wp, bp):
    return pl.pallas_call(
        _post_body,
        grid=(_GRID,),
        in_specs=[
            pl.BlockSpec((NC, _BN, TBL), lambda i: (0, i, 0)),
            pl.BlockSpec((1, HID), lambda i: (0, 0)),
            pl.BlockSpec((_BN, 1), lambda i: (i, 0)),
            pl.BlockSpec((HID, 1), lambda i: (0, 0)),
            pl.BlockSpec((1, 1), lambda i: (0, 0)),
        ],
        out_specs=pl.BlockSpec((NG, 1), lambda i: (0, 0)),
        out_shape=jax.ShapeDtypeStruct((NG, 1), jnp.float32),
        scratch_shapes=[
            pltpu.VMEM((NG, NG), jnp.float32),
            pltpu.VMEM((NG, 1), jnp.float32),
        ],
    )(part, bias2.reshape(1, HID), batch.reshape(N, 1), Wp, bp.reshape(1, 1))


def kernel(x, edge_index, batch, Wl1, bl1, Wr1, br1, att1, bias1,
           Wl2, bl2, Wr2, br2, att2, bias2, Wp, bp):
    src = edge_index[0]
    dst = edge_index[1]

    src2 = src.reshape(NCHUNK, CH)
    dst2 = dst.reshape(NCHUNK, CH)
    xlr1 = _tc_proj(x, Wl1, bl1, Wr1, br1)
    part1 = _sc_edge_kernel(HEADS, xlr1, att1.reshape(HID), src2, dst2)
    hlr2 = _tc_mid(part1, bias1, Wl2, bl2, Wr2, br2)
    part2 = _sc_edge_kernel(1, hlr2, att2.reshape(HID), src2, dst2)
    out = _tc_post(part2, bias2, batch, Wp, bp)
    return out.reshape(NG)


# speedup vs baseline: 92.8354x; 1.0013x over previous
import functools

import jax
import jax.numpy as jnp
from jax import lax
from jax.experimental import pallas as pl
from jax.experimental.pallas import tpu as pltpu
from jax.experimental.pallas import tpu_sc as plsc

N = 10000
E = 320000
DIN = 128
HID = 64
HEADS = 4
NG = 64

NC = 2
NS = 16
NW = NC * NS
L = 16

CH = 80
NCHUNK = E // CH
NCHT = NCHUNK // NW
TBL = 128
NP_ = 10240
RPT = NP_ // NS
ZR = 16

_BN = 400
_GRID = N // _BN


def _proj_body(x_ref, w_ref, b_ref, xlr_ref):
    xlr_ref[...] = (jnp.dot(x_ref[...], w_ref[...], preferred_element_type=jnp.float32)
                    + b_ref[...])


def _tc_proj(x, Wl, bl, Wr, br):
    din = x.shape[1]
    w = jnp.concatenate([Wl, Wr], axis=1)
    b = jnp.concatenate([bl, br]).reshape(1, 2 * HID)
    return pl.pallas_call(
        _proj_body,
        grid=(_GRID,),
        in_specs=[
            pl.BlockSpec((_BN, din), lambda i: (i, 0)),
            pl.BlockSpec((din, 2 * HID), lambda i: (0, 0)),
            pl.BlockSpec((1, 2 * HID), lambda i: (0, 0)),
        ],
        out_specs=pl.BlockSpec((_BN, 2 * HID), lambda i: (i, 0)),
        out_shape=jax.ShapeDtypeStruct((N, 2 * HID), jnp.float32),
    )(x, w, b)


def _sc_edge_kernel(nheads, xlr, att, src2, dst2):
    mesh = plsc.VectorSubcoreMesh(core_axis_name="c", subcore_axis_name="s")

    @functools.partial(
        pl.kernel,
        mesh=mesh,
        out_type=jax.ShapeDtypeStruct((NC, NP_, TBL), jnp.float32),
        scratch_types=[
            pltpu.VMEM_SHARED((NP_, TBL), jnp.float32),
            pltpu.VMEM((ZR, TBL), jnp.float32),
            pltpu.VMEM((CH,), jnp.int32),
            pltpu.VMEM((CH,), jnp.int32),
            pltpu.VMEM((CH,), jnp.int32),
            pltpu.VMEM((CH,), jnp.int32),
            pltpu.VMEM((CH,), jnp.int32),
            pltpu.VMEM((CH,), jnp.int32),
            pltpu.VMEM((CH,), jnp.int32),
            pltpu.VMEM((CH,), jnp.int32),
            pltpu.VMEM((CH, TBL), jnp.float32),
            pltpu.VMEM((CH, TBL), jnp.float32),
            pltpu.VMEM((CH, TBL), jnp.float32),
            pltpu.VMEM((CH, TBL), jnp.float32),
            pltpu.VMEM((HID,), jnp.float32),
        ] + [pltpu.SemaphoreType.DMA] * 15,
    )
    def k(xlr_hbm, att_hbm, src_hbm, dst_hbm, part_hbm,
          tbl_sh, zbuf, si0, si1, si2, si3, di0, di1, di2, di3,
          gs0, gs1, gd0, gd1, attv,
          zsem, ssi0, ssi1, ssi2, ssi3, sdi0, sdi1, sdi2, sdi3,
          sgs0, sgs1, sgd0, sgd1, ssc0, ssc1):
        cid = lax.axis_index("c")
        sid = lax.axis_index("s")
        wid = sid * NC + cid
        c0 = wid * NCHT

        SI = (si0, si1, si2, si3)
        DI = (di0, di1, di2, di3)
        GS = (gs0, gs1)
        GD = (gd0, gd1)
        SSI = (ssi0, ssi1, ssi2, ssi3)
        SDI = (sdi0, sdi1, sdi2, sdi3)
        SGS = (sgs0, sgs1)
        SGD = (sgd0, sgd1)
        SSC = (ssc0, ssc1)

        zero = jnp.zeros((L,), jnp.float32)

        def zrow(r, carry):
            for j in range(TBL // L):
                zbuf[r, pl.ds(j * L, L)] = zero
            return carry

        lax.fori_loop(0, ZR, zrow, 0)
        for r in range(RPT // ZR // 8):
            zcps = [pltpu.async_copy(
                        zbuf, tbl_sh.at[pl.ds(sid * RPT + (r * 8 + q) * ZR, ZR)], zsem)
                    for q in range(8)]
            for cp in zcps:
                cp.wait()
        pltpu.sync_copy(att_hbm, attv)
        plsc.subcore_barrier()

        iot = lax.iota(jnp.int32, L)
        avs = [attv[pl.ds(h * L, L)] for h in range(HEADS)]

        gdn = lax.GatherDimensionNumbers(
            offset_dims=(), collapsed_slice_dims=(0,), start_index_map=(0,))
        perm = [(iot ^ (1 << b))[:, None] for b in range(4)]

        def _allsum(v):
            for b in range(4):
                v = v + lax.gather(v, perm[b], gdn, slice_sizes=(1,),
                                   mode=lax.GatherScatterMode.PROMISE_IN_BOUNDS)
            return v

        if nheads > 1:
            def edge_body_for(gsrc, gdst):
                def edge_body(e, c2):
                    dvec = jnp.zeros((L,), jnp.float32)
                    for h in range(HEADS):
                        s = gsrc[e, pl.ds(h * L, L)]
                        d = gdst[e, pl.ds(HID + h * L, L)]
                        eh = s + d
                        eh = jnp.maximum(eh, 0.2 * eh)
                        pb = jnp.exp(_allsum(eh * avs[h]))
                        gsrc[e, pl.ds(h * L, L)] = pb * s
                        dvec = jnp.where(iot == h, pb, dvec)
                    gsrc[e, pl.ds(HID, L)] = dvec
                    return c2
                return edge_body
        else:
            def edge_body_for(gsrc, gdst):
                def edge_body(e, c2):
                    svecs = []
                    am = jnp.zeros((L,), jnp.float32)
                    for h in range(HEADS):
                        s = gsrc[e, pl.ds(h * L, L)]
                        d = gdst[e, pl.ds(HID + h * L, L)]
                        eh = s + d
                        eh = jnp.maximum(eh, 0.2 * eh)
                        am = am + eh * avs[h]
                        svecs.append(s)
                    pb = jnp.exp(_allsum(am))
                    for h in range(HEADS):
                        gsrc[e, pl.ds(h * L, L)] = pb * svecs[h]
                    gsrc[e, pl.ds(HID, L)] = jnp.where(iot == 0, pb, zero)
                    return c2
                return edge_body

        pltpu.async_copy(src_hbm.at[c0], SI[0], SSI[0])
        pltpu.async_copy(dst_hbm.at[c0], DI[0], SDI[0])
        pltpu.async_copy(src_hbm.at[c0 + 1], SI[1], SSI[1])
        pltpu.async_copy(dst_hbm.at[c0 + 1], DI[1], SDI[1])
        pltpu.make_async_copy(src_hbm.at[c0], SI[0], SSI[0]).wait()
        pltpu.make_async_copy(dst_hbm.at[c0], DI[0], SDI[0]).wait()
        pltpu.async_copy(xlr_hbm.at[SI[0]], GS[0], SGS[0])
        pltpu.async_copy(xlr_hbm.at[DI[0]], GD[0], SGD[0])

        def quad_body(p, carry):
            for b in range(4):
                S, T = b % 2, 1 - b % 2
                i4 = b
                j4 = (b + 1) % 4
                n4 = (b + 2) % 4
                w4 = (b + 3) % 4
                c = 4 * p + b

                @pl.when(c < NCHT)
                def _():
                    @pl.when(c + 1 < NCHT)
                    def _():
                        @pl.when(c >= 1)
                        def _():
                            pltpu.make_async_copy(
                                GS[T], tbl_sh.at[DI[w4]], SSC[T]).wait()
                        pltpu.make_async_copy(src_hbm.at[c0 + c + 1], SI[j4], SSI[j4]).wait()
                        pltpu.make_async_copy(dst_hbm.at[c0 + c + 1], DI[j4], SDI[j4]).wait()
                        pltpu.async_copy(xlr_hbm.at[SI[j4]], GS[T], SGS[T])
                        pltpu.async_copy(xlr_hbm.at[DI[j4]], GD[T], SGD[T])

                    pltpu.make_async_copy(xlr_hbm.at[SI[i4]], GS[S], SGS[S]).wait()
                    pltpu.make_async_copy(xlr_hbm.at[DI[i4]], GD[S], SGD[S]).wait()

                    @pl.when(c + 2 < NCHT)
                    def _():
                        pltpu.async_copy(src_hbm.at[c0 + c + 2], SI[n4], SSI[n4])
                        pltpu.async_copy(dst_hbm.at[c0 + c + 2], DI[n4], SDI[n4])

                    plsc.parallel_loop(0, CH, 1, unroll=4, carry=jnp.int32(0))(
                        edge_body_for(GS[S], GD[S]))
                    pltpu.async_copy(GS[S], tbl_sh.at[DI[i4]], SSC[S], add=True)
            return carry

        lax.fori_loop(0, (NCHT + 3) // 4, quad_body, 0)

        pltpu.make_async_copy(GS[(NCHT - 2) % 2], tbl_sh.at[DI[(NCHT - 2) % 4]],
                              SSC[(NCHT - 2) % 2]).wait()
        pltpu.make_async_copy(GS[(NCHT - 1) % 2], tbl_sh.at[DI[(NCHT - 1) % 4]],
                              SSC[(NCHT - 1) % 2]).wait()
        plsc.subcore_barrier()

        pltpu.sync_copy(tbl_sh.at[pl.ds(sid * RPT, RPT)],
                        part_hbm.at[cid, pl.ds(sid * RPT, RPT)])

    return k(xlr, att, src2, dst2)


def _mid_body(part_ref, b1_ref, wl_ref, bl_ref, hlr_ref):
    p0 = part_ref[0]
    p1 = part_ref[1]
    acc = p0[:, :HID] + p1[:, :HID]
    den = p0[:, HID:HID + HEADS] + p1[:, HID:HID + HEADS]
    hsel = (lax.broadcasted_iota(jnp.int32, (HEADS, HID), 1) // (HID // HEADS)
            == lax.broadcasted_iota(jnp.int32, (HEADS, HID), 0)).astype(jnp.float32)
    dene = jnp.dot(den, hsel, preferred_element_type=jnp.float32)
    h = jnp.maximum(acc / (dene + 1e-16) + b1_ref[...], 0.0)
    hlr_ref[...] = jnp.dot(h, wl_ref[...], preferred_element_type=jnp.float32) + bl_ref[...]


def _tc_mid(part, bias1, Wl2, bl2, Wr2, br2):
    w = jnp.concatenate([Wl2, Wr2], axis=1)
    b = jnp.concatenate([bl2, br2]).reshape(1, 2 * HID)
    return pl.pallas_call(
        _mid_body,
        grid=(_GRID,),
        in_specs=[
            pl.BlockSpec((NC, _BN, TBL), lambda i: (0, i, 0)),
            pl.BlockSpec((1, HID), lambda i: (0, 0)),
            pl.BlockSpec((HID, 2 * HID), lambda i: (0, 0)),
            pl.BlockSpec((1, 2 * HID), lambda i: (0, 0)),
        ],
        out_specs=pl.BlockSpec((_BN, 2 * HID), lambda i: (i, 0)),
        out_shape=jax.ShapeDtypeStruct((N, 2 * HID), jnp.float32),
    )(part, bias1.reshape(1, HID), w, b)


def _post_body(part_ref, b2_ref, batch_ref, wp_ref, bp_ref, out_ref,
               pooled_s, counts_s):
    i = pl.program_id(0)
    p0 = part_ref[0]
    p1 = part_ref[1]
    den = p0[:, HID:HID + 1] + p1[:, HID:HID + 1]
    h2 = jnp.maximum((p0[:, :HID] + p1[:, :HID]) / (den + 1e-16) + b2_ref[...], 0.0)
    bcol = batch_ref[...]
    P = (bcol == lax.broadcasted_iota(jnp.int32, (_BN, NG), 1)).astype(jnp.float32)
    pp = lax.dot_general(P, h2, (((0,), (0,)), ((), ())),
                         preferred_element_type=jnp.float32)
    cc = lax.dot_general(P, jnp.ones((_BN, 1), jnp.float32), (((0,), (0,)), ((), ())),
                         preferred_element_type=jnp.float32)

    @pl.when(i == 0)
    def _():
        pooled_s[...] = pp
        counts_s[...] = cc

    @pl.when(i > 0)
    def _():
        pooled_s[...] += pp
        counts_s[...] += cc

    @pl.when(i == _GRID - 1)
    def _():
        pooled = pooled_s[...] / jnp.maximum(counts_s[...], 1.0)
        out_ref[...] = jnp.dot(pooled, wp_ref[...], preferred_element_type=jnp.float32) + bp_ref[...]


def _tc_post(part, bias2, batch, Wp, bp):
    return pl.pallas_call(
        _post_body,
        grid=(_GRID,),
        in_specs=[
            pl.BlockSpec((NC, _BN, TBL), lambda i: (0, i, 0)),
            pl.BlockSpec((1, HID), lambda i: (0, 0)),
            pl.BlockSpec((_BN, 1), lambda i: (i, 0)),
            pl.BlockSpec((HID, 1), lambda i: (0, 0)),
            pl.BlockSpec((1, 1), lambda i: (0, 0)),
        ],
        out_specs=pl.BlockSpec((NG, 1), lambda i: (0, 0)),
        out_shape=jax.ShapeDtypeStruct((NG, 1), jnp.float32),
        scratch_shapes=[
            pltpu.VMEM((NG, NG), jnp.float32),
            pltpu.VMEM((NG, 1), jnp.float32),
        ],
    )(part, bias2.reshape(1, HID), batch.reshape(N, 1), Wp, bp.reshape(1, 1))


def kernel(x, edge_index, batch, Wl1, bl1, Wr1, br1, att1, bias1,
           Wl2, bl2, Wr2, br2, att2, bias2, Wp, bp):
    src = edge_index[0]
    dst = edge_index[1]

    src2 = src.reshape(NCHUNK, CH)
    dst2 = dst.reshape(NCHUNK, CH)
    xlr1 = _tc_proj(x, Wl1, bl1, Wr1, br1)
    part1 = _sc_edge_kernel(HEADS, xlr1, att1.reshape(HID), src2, dst2)
    hlr2 = _tc_mid(part1, bias1, Wl2, bl2, Wr2, br2)
    part2 = _sc_edge_kernel(1, hlr2, att2.reshape(HID), src2, dst2)
    out = _tc_post(part2, bias2, batch, Wp, bp)
    return out.reshape(NG)
